# baseline (speedup 1.0000x reference)
import functools

import jax
import jax.numpy as jnp
from jax import lax
from jax.experimental import pallas as pl
from jax.experimental.pallas import tpu as pltpu
from jax.experimental.pallas import tpu_sc as plsc

N_RES = 4096
N_IN = 512
C_TOT = N_RES + N_IN
NW = 32
LANES = 16

B3 = 288
CAP = 96
ROW = 2 * CAP
RND = 8192
ROUNDS = 8
N_PAD3 = NW * ROUNDS * RND
PAD_IDX = C_TOT * N_RES
SCR_WORDS = B3 * NW * ROUNDS * ROW
BPT = B3 // NW


def _mesh():
    return plsc.VectorSubcoreMesh(core_axis_name="c", subcore_axis_name="s")


def _partition(comb):
    @functools.partial(
        pl.kernel,
        mesh=_mesh(),
        out_type=jax.ShapeDtypeStruct((SCR_WORDS,), jnp.int32),
        scratch_types=[
            pltpu.VMEM((B3 * ROW,), jnp.int32),
            pltpu.VMEM((304,), jnp.int32),
            pltpu.VMEM((2 * RND,), jnp.int32),
            pltpu.VMEM((2 * RND,), jnp.int32),
            pltpu.SemaphoreType.DMA,
            pltpu.SemaphoreType.DMA,
            pltpu.SemaphoreType.DMA,
        ],
        compiler_params=pltpu.CompilerParams(needs_layout_passes=False),
    )
    def k(comb_hbm, scr_hbm, stage, cnt, buf0, buf1, sem0, sem1, fsem):
        wid = lax.axis_index("s") * 2 + lax.axis_index("c")
        zeros16 = jnp.zeros((LANES,), jnp.int32)
        ones16 = jnp.ones((LANES,), jnp.int32)
        bufs, sems = (buf0, buf1), (sem0, sem1)

        def zero_stage(i, c):
            stage[pl.ds(i * LANES, LANES)] = zeros16
            return c

        lax.fori_loop(0, B3 * ROW // LANES, zero_stage, 0)

        def start(rr, bb):
            chunk = wid * ROUNDS + rr
            pltpu.async_copy(
                comb_hbm.at[pl.ds(chunk * 2 * RND, 2 * RND)], bufs[bb],
                sems[bb])

        def wait_stream(bb):
            pltpu.make_async_copy(
                comb_hbm.at[pl.ds(0, 2 * RND)], bufs[bb], sems[bb]).wait()

        def drain_flush(i, c):
            pltpu.make_async_copy(
                stage.at[pl.ds(0, ROW)], scr_hbm.at[pl.ds(0, ROW)],
                fsem).wait()
            return c

        start(0, 0)
        start(1, 1)
        for r in range(ROUNDS):
            bb = r % 2
            if r > 0:
                lax.fori_loop(0, B3, drain_flush, 0)

                def zero_vals(bk, c):
                    for j in range(CAP // LANES):
                        stage[pl.ds(bk * ROW + CAP + j * LANES, LANES)] = (
                            zeros16)
                    return c

                lax.fori_loop(0, B3, zero_vals, 0)

            def zero_cnt(i, c):
                cnt[pl.ds(i * LANES, LANES)] = zeros16
                return c

            lax.fori_loop(0, 304 // LANES, zero_cnt, 0)
            wait_stream(bb)
            buf = bufs[bb]

            def vec(j, c):
                iv = buf[pl.ds(j * LANES, LANES)]
                vvb = buf[pl.ds(RND + j * LANES, LANES)]
                bkt = lax.shift_right_logical(iv, 16)
                loc = lax.bitwise_and(iv, jnp.int32(0xFFFF))
                rank = plsc.scan_count(bkt)[0]
                cnts = plsc.load_gather(cnt, [bkt])
                slot = cnts + rank - 1
                valid = (bkt < B3) & (slot < CAP)
                addr = jnp.where(valid, bkt * ROW + slot, 0)
                plsc.store_scatter(stage, [addr], loc, mask=valid)
                plsc.store_scatter(stage, [addr + CAP], vvb, mask=valid)
                plsc.addupdate_scatter(cnt, [bkt], ones16, mask=valid)
                return c

            lax.fori_loop(0, RND // LANES, vec, 0, unroll=2)
            if r + 2 < ROUNDS:
                start(r + 2, bb)

            def flush(bk, c):
                dst = ((bk * NW + wid) * ROUNDS + r) * ROW
                pltpu.async_copy(
                    stage.at[pl.ds(bk * ROW, ROW)],
                    scr_hbm.at[pl.ds(dst, ROW)], fsem)
                return c

            lax.fori_loop(0, B3, flush, 0)
        lax.fori_loop(0, B3, drain_flush, 0)

    return k(comb)


def _scatter_dense(scr, zslab):
    seg_half = NW * ROUNDS * ROW // 2

    @functools.partial(
        pl.kernel,
        mesh=_mesh(),
        out_type=jax.ShapeDtypeStruct((C_TOT * N_RES,), jnp.float32),
        scratch_types=[
            pltpu.VMEM((16 * N_RES,), jnp.float32),
            pltpu.VMEM((seg_half,), jnp.int32),
            pltpu.VMEM((seg_half,), jnp.int32),
            pltpu.SemaphoreType.DMA,
            pltpu.SemaphoreType.DMA,
        ],
        compiler_params=pltpu.CompilerParams(needs_layout_passes=False),
    )
    def k(scr_hbm, z_hbm, w_hbm, acc, sb0, sb1, sem0, sem1):
        wid = lax.axis_index("s") * 2 + lax.axis_index("c")
        sbufs, sems = (sb0, sb1), (sem0, sem1)
        for kb in range(BPT):
            b = wid * BPT + kb
            base = b * 2 * seg_half
            for h in range(2):
                pltpu.async_copy(
                    scr_hbm.at[pl.ds(base + h * seg_half, seg_half)],
                    sbufs[h], sems[h])
            pltpu.sync_copy(z_hbm, acc)
            for h in range(2):
                sbuf = sbufs[h]
                pltpu.make_async_copy(
                    scr_hbm.at[pl.ds(0, seg_half)], sbuf, sems[h]).wait()

                def seg(s, c):
                    for j in range(CAP // LANES):
                        locv = sbuf[pl.ds(s * ROW + j * LANES, LANES)]
                        vvb = sbuf[pl.ds(s * ROW + CAP + j * LANES, LANES)]
                        m = vvb != 0
                        plsc.addupdate_scatter(
                            acc, [locv], plsc.bitcast(vvb, jnp.float32),
                            mask=m)
                    return c

                lax.fori_loop(0, seg_half // ROW, seg, 0)
            pltpu.sync_copy(acc, w_hbm.at[pl.ds(b * 16 * N_RES, 16 * N_RES)])

    return k(scr, zslab)


def _erf(x):
    ax = jnp.abs(x)
    t = 1.0 / (1.0 + 0.3275911 * ax)
    poly = t * (0.254829592 + t * (-0.284496736 + t * (
        1.421413741 + t * (-1.453152027 + t * 1.061405429))))
    y = 1.0 - poly * jnp.exp(-ax * ax)
    return jnp.where(x < 0, -y, y)


BM = 512
BN = 512


def _mm_kernel(a_ref, w_ref, b_ref, o_ref):
    z = lax.dot_general(
        a_ref[...], w_ref[...], (((1,), (0,)), ((), ())),
        precision=lax.Precision.HIGHEST,
        preferred_element_type=jnp.float32)
    o_ref[...] = _erf(z + b_ref[...])


def _matmul_erf(a, w, bias2):
    m = a.shape[0]
    return pl.pallas_call(
        _mm_kernel,
        grid=(m // BM, N_RES // BN),
        in_specs=[
            pl.BlockSpec((BM, C_TOT), lambda i, j: (i, 0)),
            pl.BlockSpec((C_TOT, BN), lambda i, j: (0, j)),
            pl.BlockSpec((1, BN), lambda i, j: (0, j)),
        ],
        out_specs=pl.BlockSpec((BM, BN), lambda i, j: (i, j)),
        out_shape=jax.ShapeDtypeStruct((m, N_RES), jnp.float32),
        compiler_params=pltpu.CompilerParams(
            dimension_semantics=("parallel", "parallel")),
    )(a, w, bias2)


def kernel(state, x, res_vals, res_rows, res_cols, res_bias,
           in_vals, in_rows, in_cols):
    in_rows = in_rows.astype(jnp.int32)
    in_cols = in_cols.astype(jnp.int32)
    res_rows = res_rows.astype(jnp.int32)
    res_cols = res_cols.astype(jnp.int32)
    idx = jnp.concatenate([
        in_cols * N_RES + in_rows,
        (res_cols + N_IN) * N_RES + res_rows,
    ])
    vals = jnp.concatenate([in_vals, res_vals])
    n = idx.shape[0]
    idx = jnp.pad(idx, (0, N_PAD3 - n), constant_values=PAD_IDX)
    vals = jnp.pad(vals, (0, N_PAD3 - n))
    nchunks = NW * ROUNDS
    idx_c = idx.reshape(RND, nchunks).T
    val_c = lax.bitcast_convert_type(vals, jnp.int32).reshape(RND, nchunks).T
    comb = jnp.stack([idx_c, val_c], axis=1).reshape(-1)
    zslab = jnp.zeros((16 * N_RES,), jnp.float32)

    scr = _partition(comb)
    w = _scatter_dense(scr, zslab).reshape(C_TOT, N_RES)

    a = jnp.concatenate([x, state], axis=1)
    bias2 = res_bias.reshape(1, N_RES)
    return _matmul_erf(a, w, bias2)


# speedup vs baseline: 1.0657x; 1.0657x over previous
import functools

import jax
import jax.numpy as jnp
from jax import lax
from jax.experimental import pallas as pl
from jax.experimental.pallas import tpu as pltpu
from jax.experimental.pallas import tpu_sc as plsc

N_RES = 4096
N_IN = 512
C_TOT = N_RES + N_IN
NW = 32
LANES = 16

B3 = 288
CAP = 64
ROW = 2 * CAP
RND = 8192
ROUNDS = 8
N_PAD3 = NW * ROUNDS * RND
PAD_IDX = C_TOT * N_RES
SCR_WORDS = B3 * NW * ROUNDS * ROW
BPT = B3 // NW


def _mesh():
    return plsc.VectorSubcoreMesh(core_axis_name="c", subcore_axis_name="s")


def _partition(comb):
    @functools.partial(
        pl.kernel,
        mesh=_mesh(),
        out_type=jax.ShapeDtypeStruct((SCR_WORDS,), jnp.int32),
        scratch_types=[
            pltpu.VMEM((B3 * ROW,), jnp.int32),
            pltpu.VMEM((304,), jnp.int32),
            pltpu.VMEM((2 * RND,), jnp.int32),
            pltpu.VMEM((2 * RND,), jnp.int32),
            pltpu.SemaphoreType.DMA,
            pltpu.SemaphoreType.DMA,
            pltpu.SemaphoreType.DMA,
        ],
        compiler_params=pltpu.CompilerParams(needs_layout_passes=False),
    )
    def k(comb_hbm, scr_hbm, stage, cnt, buf0, buf1, sem0, sem1, fsem):
        wid = lax.axis_index("s") * 2 + lax.axis_index("c")
        zeros16 = jnp.zeros((LANES,), jnp.int32)
        ones16 = jnp.ones((LANES,), jnp.int32)
        bufs, sems = (buf0, buf1), (sem0, sem1)

        def zero_stage(i, c):
            stage[pl.ds(i * LANES, LANES)] = zeros16
            return c

        lax.fori_loop(0, B3 * ROW // LANES, zero_stage, 0)

        def start(rr, bb):
            chunk = wid * ROUNDS + rr
            pltpu.async_copy(
                comb_hbm.at[pl.ds(chunk * 2 * RND, 2 * RND)], bufs[bb],
                sems[bb])

        def wait_stream(bb):
            pltpu.make_async_copy(
                comb_hbm.at[pl.ds(0, 2 * RND)], bufs[bb], sems[bb]).wait()

        def drain_flush(i, c):
            pltpu.make_async_copy(
                stage.at[pl.ds(0, ROW)], scr_hbm.at[pl.ds(0, ROW)],
                fsem).wait()
            return c

        start(0, 0)
        start(1, 1)
        for r in range(ROUNDS):
            bb = r % 2
            if r > 0:
                lax.fori_loop(0, B3, drain_flush, 0)

                def zero_vals(bk, c):
                    for j in range(CAP // LANES):
                        stage[pl.ds(bk * ROW + CAP + j * LANES, LANES)] = (
                            zeros16)
                    return c

                lax.fori_loop(0, B3, zero_vals, 0)

            def zero_cnt(i, c):
                cnt[pl.ds(i * LANES, LANES)] = zeros16
                return c

            lax.fori_loop(0, 304 // LANES, zero_cnt, 0)
            wait_stream(bb)
            buf = bufs[bb]

            def vec(j, c):
                iv = buf[pl.ds(j * LANES, LANES)]
                vvb = buf[pl.ds(RND + j * LANES, LANES)]
                bkt = lax.shift_right_logical(iv, 16)
                loc = lax.bitwise_and(iv, jnp.int32(0xFFFF))
                rank = plsc.scan_count(bkt)[0]
                cnts = plsc.load_gather(cnt, [bkt])
                slot = cnts + rank - 1
                valid = (bkt < B3) & (slot < CAP)
                addr = jnp.where(valid, bkt * ROW + slot, 0)
                plsc.store_scatter(stage, [addr], loc, mask=valid)
                plsc.store_scatter(stage, [addr + CAP], vvb, mask=valid)
                plsc.addupdate_scatter(cnt, [bkt], ones16, mask=valid)
                return c

            lax.fori_loop(0, RND // LANES, vec, 0, unroll=2)
            if r + 2 < ROUNDS:
                start(r + 2, bb)

            def flush(bk, c):
                dst = ((bk * NW + wid) * ROUNDS + r) * ROW
                pltpu.async_copy(
                    stage.at[pl.ds(bk * ROW, ROW)],
                    scr_hbm.at[pl.ds(dst, ROW)], fsem)
                return c

            lax.fori_loop(0, B3, flush, 0)
        lax.fori_loop(0, B3, drain_flush, 0)

    return k(comb)


def _scatter_dense(scr, zslab):
    seg_half = NW * ROUNDS * ROW // 2

    @functools.partial(
        pl.kernel,
        mesh=_mesh(),
        out_type=jax.ShapeDtypeStruct((C_TOT * N_RES,), jnp.float32),
        scratch_types=[
            pltpu.VMEM((16 * N_RES,), jnp.float32),
            pltpu.VMEM((seg_half,), jnp.int32),
            pltpu.VMEM((seg_half,), jnp.int32),
            pltpu.SemaphoreType.DMA,
            pltpu.SemaphoreType.DMA,
        ],
        compiler_params=pltpu.CompilerParams(needs_layout_passes=False),
    )
    def k(scr_hbm, z_hbm, w_hbm, acc, sb0, sb1, sem0, sem1):
        wid = lax.axis_index("s") * 2 + lax.axis_index("c")
        sbufs, sems = (sb0, sb1), (sem0, sem1)
        for kb in range(BPT):
            b = wid * BPT + kb
            base = b * 2 * seg_half
            for h in range(2):
                pltpu.async_copy(
                    scr_hbm.at[pl.ds(base + h * seg_half, seg_half)],
                    sbufs[h], sems[h])
            pltpu.sync_copy(z_hbm, acc)
            for h in range(2):
                sbuf = sbufs[h]
                pltpu.make_async_copy(
                    scr_hbm.at[pl.ds(0, seg_half)], sbuf, sems[h]).wait()

                def seg(s, c):
                    for j in range(CAP // LANES):
                        locv = sbuf[pl.ds(s * ROW + j * LANES, LANES)]
                        vvb = sbuf[pl.ds(s * ROW + CAP + j * LANES, LANES)]
                        m = vvb != 0
                        plsc.addupdate_scatter(
                            acc, [locv], plsc.bitcast(vvb, jnp.float32),
                            mask=m)
                    return c

                lax.fori_loop(0, seg_half // ROW, seg, 0)
            pltpu.sync_copy(acc, w_hbm.at[pl.ds(b * 16 * N_RES, 16 * N_RES)])

    return k(scr, zslab)


def _erf(x):
    ax = jnp.abs(x)
    t = 1.0 / (1.0 + 0.3275911 * ax)
    poly = t * (0.254829592 + t * (-0.284496736 + t * (
        1.421413741 + t * (-1.453152027 + t * 1.061405429))))
    y = 1.0 - poly * jnp.exp(-ax * ax)
    return jnp.where(x < 0, -y, y)


BM = 512
BN = 512


def _mm_kernel(a_ref, w_ref, b_ref, o_ref):
    z = lax.dot_general(
        a_ref[...], w_ref[...], (((1,), (0,)), ((), ())),
        precision=lax.Precision.HIGHEST,
        preferred_element_type=jnp.float32)
    o_ref[...] = _erf(z + b_ref[...])


def _matmul_erf(a, w, bias2):
    m = a.shape[0]
    return pl.pallas_call(
        _mm_kernel,
        grid=(m // BM, N_RES // BN),
        in_specs=[
            pl.BlockSpec((BM, C_TOT), lambda i, j: (i, 0)),
            pl.BlockSpec((C_TOT, BN), lambda i, j: (0, j)),
            pl.BlockSpec((1, BN), lambda i, j: (0, j)),
        ],
        out_specs=pl.BlockSpec((BM, BN), lambda i, j: (i, j)),
        out_shape=jax.ShapeDtypeStruct((m, N_RES), jnp.float32),
        compiler_params=pltpu.CompilerParams(
            dimension_semantics=("parallel", "parallel")),
    )(a, w, bias2)


def kernel(state, x, res_vals, res_rows, res_cols, res_bias,
           in_vals, in_rows, in_cols):
    in_rows = in_rows.astype(jnp.int32)
    in_cols = in_cols.astype(jnp.int32)
    res_rows = res_rows.astype(jnp.int32)
    res_cols = res_cols.astype(jnp.int32)
    idx = jnp.concatenate([
        in_cols * N_RES + in_rows,
        (res_cols + N_IN) * N_RES + res_rows,
    ])
    vals = jnp.concatenate([in_vals, res_vals])
    n = idx.shape[0]
    idx = jnp.pad(idx, (0, N_PAD3 - n), constant_values=PAD_IDX)
    vals = jnp.pad(vals, (0, N_PAD3 - n))
    nchunks = NW * ROUNDS
    idx_c = idx.reshape(RND, nchunks).T
    val_c = lax.bitcast_convert_type(vals, jnp.int32).reshape(RND, nchunks).T
    comb = jnp.stack([idx_c, val_c], axis=1).reshape(-1)
    zslab = jnp.zeros((16 * N_RES,), jnp.float32)

    scr = _partition(comb)
    w = _scatter_dense(scr, zslab).reshape(C_TOT, N_RES)

    a = jnp.concatenate([x, state], axis=1)
    bias2 = res_bias.reshape(1, N_RES)
    return _matmul_erf(a, w, bias2)


# speedup vs baseline: 1.1318x; 1.0621x over previous
import functools

import jax
import jax.numpy as jnp
from jax import lax
from jax.experimental import pallas as pl
from jax.experimental.pallas import tpu as pltpu
from jax.experimental.pallas import tpu_sc as plsc

N_RES = 4096
N_IN = 512
C_TOT = N_RES + N_IN
NW = 32
LANES = 16

B3 = 288
CAP = 64
ROW = 2 * CAP
RND = 8192
ROUNDS = 8
N_PAD3 = NW * ROUNDS * RND
PAD_IDX = C_TOT * N_RES
SCR_WORDS = B3 * NW * ROUNDS * ROW
BPT = B3 // NW


def _mesh():
    return plsc.VectorSubcoreMesh(core_axis_name="c", subcore_axis_name="s")


def _partition(comb):
    @functools.partial(
        pl.kernel,
        mesh=_mesh(),
        out_type=jax.ShapeDtypeStruct((SCR_WORDS,), jnp.int32),
        scratch_types=[
            pltpu.VMEM((B3 * ROW,), jnp.int32),
            pltpu.VMEM((304,), jnp.int32),
            pltpu.VMEM((2 * RND,), jnp.int32),
            pltpu.VMEM((2 * RND,), jnp.int32),
            pltpu.SemaphoreType.DMA,
            pltpu.SemaphoreType.DMA,
            pltpu.SemaphoreType.DMA,
        ],
        compiler_params=pltpu.CompilerParams(needs_layout_passes=False),
    )
    def k(comb_hbm, scr_hbm, stage, cnt, buf0, buf1, sem0, sem1, fsem):
        wid = lax.axis_index("s") * 2 + lax.axis_index("c")
        zeros16 = jnp.zeros((LANES,), jnp.int32)
        ones16 = jnp.ones((LANES,), jnp.int32)
        bufs, sems = (buf0, buf1), (sem0, sem1)

        def zero_stage(i, c):
            stage[pl.ds(i * LANES, LANES)] = zeros16
            return c

        lax.fori_loop(0, B3 * ROW // LANES, zero_stage, 0)

        def start(rr, bb):
            chunk = wid * ROUNDS + rr
            pltpu.async_copy(
                comb_hbm.at[pl.ds(chunk * 2 * RND, 2 * RND)], bufs[bb],
                sems[bb])

        def wait_stream(bb):
            pltpu.make_async_copy(
                comb_hbm.at[pl.ds(0, 2 * RND)], bufs[bb], sems[bb]).wait()

        def drain_flush(i, c):
            pltpu.make_async_copy(
                stage.at[pl.ds(0, ROW)], scr_hbm.at[pl.ds(0, ROW)],
                fsem).wait()
            return c

        start(0, 0)
        start(1, 1)
        for r in range(ROUNDS):
            bb = r % 2
            if r > 0:
                lax.fori_loop(0, B3, drain_flush, 0)

                def zero_vals(bk, c):
                    for j in range(CAP // LANES):
                        stage[pl.ds(bk * ROW + CAP + j * LANES, LANES)] = (
                            zeros16)
                    return c

                lax.fori_loop(0, B3, zero_vals, 0)

            def zero_cnt(i, c):
                cnt[pl.ds(i * LANES, LANES)] = zeros16
                return c

            lax.fori_loop(0, 304 // LANES, zero_cnt, 0)
            wait_stream(bb)
            buf = bufs[bb]

            def vec(j, c):
                iv = buf[pl.ds(j * LANES, LANES)]
                vvb = buf[pl.ds(RND + j * LANES, LANES)]
                bkt = lax.shift_right_logical(iv, 16)
                loc = lax.bitwise_and(iv, jnp.int32(0xFFFF))
                rank = plsc.scan_count(bkt)[0]
                cnts = plsc.load_gather(cnt, [bkt])
                slot = cnts + rank - 1
                valid = (bkt < B3) & (slot < CAP)
                addr = jnp.where(valid, bkt * ROW + slot, 0)
                plsc.store_scatter(stage, [addr], loc, mask=valid)
                plsc.store_scatter(stage, [addr + CAP], vvb, mask=valid)
                plsc.addupdate_scatter(cnt, [bkt], ones16, mask=valid)
                return c

            lax.fori_loop(0, RND // LANES, vec, 0, unroll=4)
            if r + 2 < ROUNDS:
                start(r + 2, bb)

            def flush(bk, c):
                dst = ((bk * NW + wid) * ROUNDS + r) * ROW
                pltpu.async_copy(
                    stage.at[pl.ds(bk * ROW, ROW)],
                    scr_hbm.at[pl.ds(dst, ROW)], fsem)
                return c

            lax.fori_loop(0, B3, flush, 0)
        lax.fori_loop(0, B3, drain_flush, 0)

    return k(comb)


def _scatter_dense(scr):
    seg_half = NW * ROUNDS * ROW // 2

    @functools.partial(
        pl.kernel,
        mesh=_mesh(),
        out_type=jax.ShapeDtypeStruct((C_TOT * N_RES,), jnp.float32),
        scratch_types=[
            pltpu.VMEM((16 * N_RES,), jnp.float32),
            pltpu.VMEM((seg_half,), jnp.int32),
            pltpu.VMEM((seg_half,), jnp.int32),
            pltpu.SemaphoreType.DMA,
            pltpu.SemaphoreType.DMA,
        ],
        compiler_params=pltpu.CompilerParams(needs_layout_passes=False),
    )
    def k(scr_hbm, w_hbm, acc, sb0, sb1, sem0, sem1):
        wid = lax.axis_index("s") * 2 + lax.axis_index("c")
        sbufs, sems = (sb0, sb1), (sem0, sem1)
        zeros16f = jnp.zeros((LANES,), jnp.float32)

        def zero_acc(i, c):
            acc[pl.ds(i * LANES, LANES)] = zeros16f
            return c

        for kb in range(BPT):
            b = wid * BPT + kb
            base = b * 2 * seg_half
            for h in range(2):
                pltpu.async_copy(
                    scr_hbm.at[pl.ds(base + h * seg_half, seg_half)],
                    sbufs[h], sems[h])
            lax.fori_loop(0, 16 * N_RES // LANES, zero_acc, 0, unroll=8)
            for h in range(2):
                sbuf = sbufs[h]
                pltpu.make_async_copy(
                    scr_hbm.at[pl.ds(0, seg_half)], sbuf, sems[h]).wait()

                def seg(s, c):
                    for j in range(CAP // LANES):
                        locv = sbuf[pl.ds(s * ROW + j * LANES, LANES)]
                        vvb = sbuf[pl.ds(s * ROW + CAP + j * LANES, LANES)]
                        m = vvb != 0
                        plsc.addupdate_scatter(
                            acc, [locv], plsc.bitcast(vvb, jnp.float32),
                            mask=m)
                    return c

                lax.fori_loop(0, seg_half // ROW, seg, 0)
            pltpu.sync_copy(acc, w_hbm.at[pl.ds(b * 16 * N_RES, 16 * N_RES)])

    return k(scr)


def _erf(x):
    ax = jnp.abs(x)
    t = 1.0 / (1.0 + 0.3275911 * ax)
    poly = t * (0.254829592 + t * (-0.284496736 + t * (
        1.421413741 + t * (-1.453152027 + t * 1.061405429))))
    y = 1.0 - poly * jnp.exp(-ax * ax)
    return jnp.where(x < 0, -y, y)


BM = 512
BN = 512


def _mm_kernel(a_ref, w_ref, b_ref, o_ref):
    z = lax.dot_general(
        a_ref[...], w_ref[...], (((1,), (0,)), ((), ())),
        precision=lax.Precision.HIGHEST,
        preferred_element_type=jnp.float32)
    o_ref[...] = _erf(z + b_ref[...])


def _matmul_erf(a, w, bias2):
    m = a.shape[0]
    return pl.pallas_call(
        _mm_kernel,
        grid=(m // BM, N_RES // BN),
        in_specs=[
            pl.BlockSpec((BM, C_TOT), lambda i, j: (i, 0)),
            pl.BlockSpec((C_TOT, BN), lambda i, j: (0, j)),
            pl.BlockSpec((1, BN), lambda i, j: (0, j)),
        ],
        out_specs=pl.BlockSpec((BM, BN), lambda i, j: (i, j)),
        out_shape=jax.ShapeDtypeStruct((m, N_RES), jnp.float32),
        compiler_params=pltpu.CompilerParams(
            dimension_semantics=("parallel", "parallel")),
    )(a, w, bias2)


def kernel(state, x, res_vals, res_rows, res_cols, res_bias,
           in_vals, in_rows, in_cols):
    in_rows = in_rows.astype(jnp.int32)
    in_cols = in_cols.astype(jnp.int32)
    res_rows = res_rows.astype(jnp.int32)
    res_cols = res_cols.astype(jnp.int32)
    idx = jnp.concatenate([
        in_cols * N_RES + in_rows,
        (res_cols + N_IN) * N_RES + res_rows,
    ])
    vals = jnp.concatenate([in_vals, res_vals])
    n = idx.shape[0]
    idx = jnp.pad(idx, (0, N_PAD3 - n), constant_values=PAD_IDX)
    vals = jnp.pad(vals, (0, N_PAD3 - n))
    nchunks = NW * ROUNDS
    idx_c = idx.reshape(RND, nchunks).T
    val_c = lax.bitcast_convert_type(vals, jnp.int32).reshape(RND, nchunks).T
    comb = jnp.stack([idx_c, val_c], axis=1).reshape(-1)
    scr = _partition(comb)
    w = _scatter_dense(scr).reshape(C_TOT, N_RES)

    a = jnp.concatenate([x, state], axis=1)
    bias2 = res_bias.reshape(1, N_RES)
    return _matmul_erf(a, w, bias2)


# speedup vs baseline: 1.1832x; 1.0454x over previous
import functools

import jax
import jax.numpy as jnp
from jax import lax
from jax.experimental import pallas as pl
from jax.experimental.pallas import tpu as pltpu
from jax.experimental.pallas import tpu_sc as plsc

N_RES = 4096
N_IN = 512
C_TOT = N_RES + N_IN
NW = 32
LANES = 16

B3 = 288
CAP = 64
ROW = 2 * CAP
RND = 8192
ROUNDS = 8
N_PAD3 = NW * ROUNDS * RND
PAD_IDX = C_TOT * N_RES
SCR_WORDS = B3 * NW * ROUNDS * ROW
BPT = B3 // NW


def _mesh():
    return plsc.VectorSubcoreMesh(core_axis_name="c", subcore_axis_name="s")


def _partition(comb):
    @functools.partial(
        pl.kernel,
        mesh=_mesh(),
        out_type=jax.ShapeDtypeStruct((SCR_WORDS,), jnp.int32),
        scratch_types=[
            pltpu.VMEM((B3 * ROW,), jnp.int32),
            pltpu.VMEM((304,), jnp.int32),
            pltpu.VMEM((2 * RND,), jnp.int32),
            pltpu.VMEM((2 * RND,), jnp.int32),
            pltpu.SemaphoreType.DMA,
            pltpu.SemaphoreType.DMA,
            pltpu.SemaphoreType.DMA,
        ],
        compiler_params=pltpu.CompilerParams(needs_layout_passes=False),
    )
    def k(comb_hbm, scr_hbm, stage, cnt, buf0, buf1, sem0, sem1, fsem):
        wid = lax.axis_index("s") * 2 + lax.axis_index("c")
        zeros16 = jnp.zeros((LANES,), jnp.int32)
        ones16 = jnp.ones((LANES,), jnp.int32)
        bufs, sems = (buf0, buf1), (sem0, sem1)

        def zero_stage(i, c):
            stage[pl.ds(i * LANES, LANES)] = zeros16
            return c

        lax.fori_loop(0, B3 * ROW // LANES, zero_stage, 0)

        def start(rr, bb):
            chunk = wid * ROUNDS + rr
            pltpu.async_copy(
                comb_hbm.at[pl.ds(chunk * 2 * RND, 2 * RND)], bufs[bb],
                sems[bb])

        def wait_stream(bb):
            pltpu.make_async_copy(
                comb_hbm.at[pl.ds(0, 2 * RND)], bufs[bb], sems[bb]).wait()

        def drain_flush(i, c):
            pltpu.make_async_copy(
                stage.at[pl.ds(0, ROW)], scr_hbm.at[pl.ds(0, ROW)],
                fsem).wait()
            return c

        start(0, 0)
        start(1, 1)
        for r in range(ROUNDS):
            bb = r % 2
            if r > 0:
                lax.fori_loop(0, B3, drain_flush, 0)

                def zero_vals(bk, c):
                    for j in range(CAP // LANES):
                        stage[pl.ds(bk * ROW + CAP + j * LANES, LANES)] = (
                            zeros16)
                    return c

                lax.fori_loop(0, B3, zero_vals, 0)

            def zero_cnt(i, c):
                cnt[pl.ds(i * LANES, LANES)] = zeros16
                return c

            lax.fori_loop(0, 304 // LANES, zero_cnt, 0)
            wait_stream(bb)
            buf = bufs[bb]

            def vec(j, c):
                iv = buf[pl.ds(j * LANES, LANES)]
                vvb = buf[pl.ds(RND + j * LANES, LANES)]
                bkt = lax.shift_right_logical(iv, 16)
                loc = lax.bitwise_and(iv, jnp.int32(0xFFFF))
                rank = plsc.scan_count(bkt)[0]
                cnts = plsc.load_gather(cnt, [bkt])
                slot = cnts + rank - 1
                valid = (bkt < B3) & (slot < CAP)
                addr = jnp.where(valid, bkt * ROW + slot, 0)
                plsc.store_scatter(stage, [addr], loc, mask=valid)
                plsc.store_scatter(stage, [addr + CAP], vvb, mask=valid)
                plsc.addupdate_scatter(cnt, [bkt], ones16, mask=valid)
                return c

            lax.fori_loop(0, RND // LANES, vec, 0, unroll=4)
            if r + 2 < ROUNDS:
                start(r + 2, bb)

            def flush(bk, c):
                dst = ((bk * NW + wid) * ROUNDS + r) * ROW
                pltpu.async_copy(
                    stage.at[pl.ds(bk * ROW, ROW)],
                    scr_hbm.at[pl.ds(dst, ROW)], fsem)
                return c

            lax.fori_loop(0, B3, flush, 0)
        lax.fori_loop(0, B3, drain_flush, 0)

    return k(comb)


def _scatter_dense(scr, bpt, base):
    seg_half = NW * ROUNDS * ROW // 2

    @functools.partial(
        pl.kernel,
        mesh=_mesh(),
        out_type=jax.ShapeDtypeStruct((NW * bpt * 16 * N_RES,), jnp.float32),
        scratch_types=[
            pltpu.VMEM((16 * N_RES,), jnp.float32),
            pltpu.VMEM((seg_half,), jnp.int32),
            pltpu.VMEM((seg_half,), jnp.int32),
            pltpu.SemaphoreType.DMA,
            pltpu.SemaphoreType.DMA,
        ],
        compiler_params=pltpu.CompilerParams(needs_layout_passes=False),
    )
    def k(scr_hbm, w_hbm, acc, sb0, sb1, sem0, sem1):
        wid = lax.axis_index("s") * 2 + lax.axis_index("c")
        sbufs, sems = (sb0, sb1), (sem0, sem1)
        zeros16f = jnp.zeros((LANES,), jnp.float32)

        def zero_acc(i, c):
            acc[pl.ds(i * LANES, LANES)] = zeros16f
            return c

        for kb in range(bpt):
            b = base + wid * bpt + kb
            src0 = b * 2 * seg_half
            for h in range(2):
                pltpu.async_copy(
                    scr_hbm.at[pl.ds(src0 + h * seg_half, seg_half)],
                    sbufs[h], sems[h])
            lax.fori_loop(0, 16 * N_RES // LANES, zero_acc, 0, unroll=8)
            for h in range(2):
                sbuf = sbufs[h]
                pltpu.make_async_copy(
                    scr_hbm.at[pl.ds(0, seg_half)], sbuf, sems[h]).wait()

                def seg(s, c):
                    for j in range(CAP // LANES):
                        locv = sbuf[pl.ds(s * ROW + j * LANES, LANES)]
                        vvb = sbuf[pl.ds(s * ROW + CAP + j * LANES, LANES)]
                        m = vvb != 0
                        plsc.addupdate_scatter(
                            acc, [locv], plsc.bitcast(vvb, jnp.float32),
                            mask=m)
                    return c

                lax.fori_loop(0, seg_half // ROW, seg, 0)
            pltpu.sync_copy(
                acc,
                w_hbm.at[pl.ds((b - base) * 16 * N_RES, 16 * N_RES)])

    return k(scr)


def _erf(x):
    ax = jnp.abs(x)
    t = 1.0 / (1.0 + 0.3275911 * ax)
    poly = t * (0.254829592 + t * (-0.284496736 + t * (
        1.421413741 + t * (-1.453152027 + t * 1.061405429))))
    y = 1.0 - poly * jnp.exp(-ax * ax)
    return jnp.where(x < 0, -y, y)


BM = 512
BN = 512


def _dot(a_ref, w_ref):
    return lax.dot_general(
        a_ref[...], w_ref[...], (((1,), (0,)), ((), ())),
        precision=lax.Precision.HIGHEST,
        preferred_element_type=jnp.float32)


def _mm_partial_kernel(a_ref, w_ref, o_ref):
    o_ref[...] = _dot(a_ref, w_ref)


def _mm_final_kernel(a_ref, w_ref, z_ref, b_ref, o_ref):
    o_ref[...] = _erf(_dot(a_ref, w_ref) + z_ref[...] + b_ref[...])


def _matmul_partial(a, w):
    m, kk = a.shape
    return pl.pallas_call(
        _mm_partial_kernel,
        grid=(m // BM, N_RES // BN),
        in_specs=[
            pl.BlockSpec((BM, kk), lambda i, j: (i, 0)),
            pl.BlockSpec((kk, BN), lambda i, j: (0, j)),
        ],
        out_specs=pl.BlockSpec((BM, BN), lambda i, j: (i, j)),
        out_shape=jax.ShapeDtypeStruct((m, N_RES), jnp.float32),
        compiler_params=pltpu.CompilerParams(
            dimension_semantics=("parallel", "parallel")),
    )(a, w)


def _matmul_final(a, w, z1, bias2):
    m, kk = a.shape
    return pl.pallas_call(
        _mm_final_kernel,
        grid=(m // BM, N_RES // BN),
        in_specs=[
            pl.BlockSpec((BM, kk), lambda i, j: (i, 0)),
            pl.BlockSpec((kk, BN), lambda i, j: (0, j)),
            pl.BlockSpec((BM, BN), lambda i, j: (i, j)),
            pl.BlockSpec((1, BN), lambda i, j: (0, j)),
        ],
        out_specs=pl.BlockSpec((BM, BN), lambda i, j: (i, j)),
        out_shape=jax.ShapeDtypeStruct((m, N_RES), jnp.float32),
        compiler_params=pltpu.CompilerParams(
            dimension_semantics=("parallel", "parallel")),
    )(a, w, z1, bias2)


def kernel(state, x, res_vals, res_rows, res_cols, res_bias,
           in_vals, in_rows, in_cols):
    in_rows = in_rows.astype(jnp.int32)
    in_cols = in_cols.astype(jnp.int32)
    res_rows = res_rows.astype(jnp.int32)
    res_cols = res_cols.astype(jnp.int32)
    idx = jnp.concatenate([
        in_cols * N_RES + in_rows,
        (res_cols + N_IN) * N_RES + res_rows,
    ])
    vals = jnp.concatenate([in_vals, res_vals])
    n = idx.shape[0]
    idx = jnp.pad(idx, (0, N_PAD3 - n), constant_values=PAD_IDX)
    vals = jnp.pad(vals, (0, N_PAD3 - n))
    nchunks = NW * ROUNDS
    idx_c = idx.reshape(RND, nchunks).T
    val_c = lax.bitcast_convert_type(vals, jnp.int32).reshape(RND, nchunks).T
    comb = jnp.stack([idx_c, val_c], axis=1).reshape(-1)
    scr = _partition(comb)
    wa = _scatter_dense(scr, 5, 0).reshape(160 * 16, N_RES)
    wb = _scatter_dense(scr, 4, 160).reshape(128 * 16, N_RES)

    a = jnp.concatenate([x, state], axis=1)
    bias2 = res_bias.reshape(1, N_RES)
    ka = 160 * 16
    z1 = _matmul_partial(a[:, :ka], wa)
    return _matmul_final(a[:, ka:], wb, z1, bias2)
